# baseline (speedup 1.0000x reference)
import functools

import jax
import jax.numpy as jnp
from jax import lax
from jax.experimental import pallas as pl
from jax.experimental.pallas import tpu as pltpu
from jax.experimental.pallas import tpu_sc as plsc

DEGREE = 64
EMBED = 128
LANES = 16
WORKERS = 16
ROWS_PER_W = DEGREE // WORKERS
BLK_ROWS = 8
GROUPS = EMBED // LANES


def _sc_l1_mean(node1, l1t, a2e):
    mesh = plsc.VectorSubcoreMesh(
        core_axis_name="c", subcore_axis_name="s", num_cores=1
    )

    @functools.partial(
        pl.kernel,
        out_type=jax.ShapeDtypeStruct((EMBED,), jnp.float32),
        mesh=mesh,
        compiler_params=pltpu.CompilerParams(needs_layout_passes=False),
        scratch_types=[
            pltpu.VMEM((LANES,), jnp.int32),
            pltpu.VMEM((BLK_ROWS, 128), jnp.int32),
            pltpu.VMEM((LANES,), jnp.int32),
            pltpu.VMEM((ROWS_PER_W, EMBED), jnp.float32),
            pltpu.VMEM((1, EMBED), jnp.float32),
            pltpu.VMEM((LANES,), jnp.int32),
            pltpu.VMEM_SHARED((1, EMBED), jnp.float32),
            pltpu.SemaphoreType.DMA,
        ],
    )
    def run(
        node_hbm, l1t_hbm, a2e_hbm, out_hbm,
        nidx_v, blk_v, ids_v, rows_v, fin_v, zidx_v, acc_s, sem,
    ):
        w = lax.axis_index("s")

        @pl.when(w == 0)
        def _():
            for g in range(GROUPS):
                fin_v[0, pl.ds(g * LANES, LANES)] = jnp.zeros(
                    (LANES,), jnp.float32
                )
            pltpu.sync_copy(fin_v, acc_s)

        pltpu.sync_copy(node_hbm, nidx_v.at[pl.ds(0, 1)])
        n = nidx_v[...][0]
        base = pl.multiple_of((n >> 7) << 7, 128)
        off = jnp.full((LANES,), n & 127, jnp.int32)
        pltpu.sync_copy(
            l1t_hbm.at[
                pl.ds((w >> 1) * BLK_ROWS, BLK_ROWS), pl.ds(base, 128)
            ],
            blk_v,
        )
        rows_idx = (lax.iota(jnp.int32, LANES) & (ROWS_PER_W - 1)) + (
            (w & 1) * ROWS_PER_W
        )
        ids_v[...] = plsc.load_gather(blk_v, [rows_idx, off])
        gather = pltpu.async_copy(
            a2e_hbm.at[ids_v.at[pl.ds(0, ROWS_PER_W)]], rows_v, sem
        )
        zidx_v[...] = jnp.zeros((LANES,), jnp.int32)
        gather.wait()
        plsc.subcore_barrier()
        pltpu.sync_copy(
            rows_v, acc_s.at[zidx_v.at[pl.ds(0, ROWS_PER_W)]], add=True
        )
        plsc.subcore_barrier()

        @pl.when(w < GROUPS)
        def _():
            pltpu.sync_copy(acc_s, fin_v)
            scale = jnp.float32(1.0 / DEGREE)
            sl = pl.ds(w * LANES, LANES)
            fin_v[0, sl] = fin_v[0, sl] * scale
            pltpu.sync_copy(fin_v.at[0].at[sl], out_hbm.at[sl])

    return run(node1, l1t, a2e)


def kernel(node, node_l1path, a2e, p2e):
    del p2e
    node1 = jnp.reshape(jnp.asarray(node, jnp.int32), (1,))
    return _sc_l1_mean(node1, node_l1path.T, a2e)


# speedup vs baseline: 1.0047x; 1.0047x over previous
import functools

import jax
import jax.numpy as jnp
from jax import lax
from jax.experimental import pallas as pl
from jax.experimental.pallas import tpu as pltpu
from jax.experimental.pallas import tpu_sc as plsc

DEGREE = 64
EMBED = 128
LANES = 16
WORKERS = 16
ROWS_PER_W = DEGREE // WORKERS
BLK_ROWS = 8
GROUPS = EMBED // LANES


def _sc_l1_mean(node1, l1t, a2e):
    mesh = plsc.VectorSubcoreMesh(
        core_axis_name="c", subcore_axis_name="s", num_cores=1
    )

    @functools.partial(
        pl.kernel,
        out_type=jax.ShapeDtypeStruct((EMBED,), jnp.float32),
        mesh=mesh,
        compiler_params=pltpu.CompilerParams(needs_layout_passes=False),
        scratch_types=[
            pltpu.VMEM((LANES,), jnp.int32),
            pltpu.VMEM((BLK_ROWS, 128), jnp.int32),
            pltpu.VMEM((LANES,), jnp.int32),
            pltpu.VMEM((ROWS_PER_W, EMBED), jnp.float32),
            pltpu.VMEM((1, EMBED), jnp.float32),
            pltpu.VMEM((LANES,), jnp.int32),
            pltpu.VMEM_SHARED((1, EMBED), jnp.float32),
            pltpu.SemaphoreType.DMA,
        ],
    )
    def run(
        node_hbm, l1t_hbm, a2e_hbm, out_hbm,
        nidx_v, blk_v, ids_v, rows_v, fin_v, zidx_v, acc_s, sem,
    ):
        w = lax.axis_index("s")

        @pl.when(w == 0)
        def _():
            for g in range(GROUPS):
                fin_v[0, pl.ds(g * LANES, LANES)] = jnp.zeros(
                    (LANES,), jnp.float32
                )
            pltpu.sync_copy(fin_v, acc_s)

        rows_idx = (lax.iota(jnp.int32, LANES) & (ROWS_PER_W - 1)) + (
            (w & 1) * ROWS_PER_W
        )
        zidx_v[...] = jnp.zeros((LANES,), jnp.int32)
        pltpu.sync_copy(node_hbm, nidx_v.at[pl.ds(0, 1)])
        n = nidx_v[...][0]
        base = pl.multiple_of((n >> 7) << 7, 128)
        off = jnp.full((LANES,), n & 127, jnp.int32)
        pltpu.sync_copy(
            l1t_hbm.at[
                pl.ds((w >> 1) * BLK_ROWS, BLK_ROWS), pl.ds(base, 128)
            ],
            blk_v,
        )
        ids_v[...] = plsc.load_gather(blk_v, [rows_idx, off])
        gather = pltpu.async_copy(
            a2e_hbm.at[ids_v.at[pl.ds(0, ROWS_PER_W)]], rows_v, sem
        )
        plsc.subcore_barrier()
        gather.wait()
        pltpu.sync_copy(
            rows_v, acc_s.at[zidx_v.at[pl.ds(0, ROWS_PER_W)]], add=True
        )
        plsc.subcore_barrier()

        @pl.when(w < GROUPS)
        def _():
            pltpu.sync_copy(acc_s, fin_v)
            scale = jnp.float32(1.0 / DEGREE)
            sl = pl.ds(w * LANES, LANES)
            fin_v[0, sl] = fin_v[0, sl] * scale
            pltpu.sync_copy(fin_v.at[0].at[sl], out_hbm.at[sl])

    return run(node1, l1t, a2e)


def kernel(node, node_l1path, a2e, p2e):
    del p2e
    node1 = jnp.reshape(jnp.asarray(node, jnp.int32), (1,))
    return _sc_l1_mean(node1, node_l1path.T, a2e)
